# baseline (speedup 1.0000x reference)
import jax
import jax.numpy as jnp
from jax import lax
from jax.experimental import pallas as pl
from jax.experimental.pallas import tpu as pltpu
from jax.experimental.pallas import tpu_sc as plsc

B = 64
A = 100000
L = 512
LANES = 16
NUM_CORES = 2
NUM_SUBCORES = 16
NW = NUM_CORES * NUM_SUBCORES
RPW = B // NW
CHUNK = 128
NCH = L // CHUNK
KCH = RPW * NCH
T = (A + 127) // 128
AP = T * 128
G = B // 8


def _red_scalar(vec, op):
    acc = vec[0]
    for i in range(1, LANES):
        acc = op(acc, vec[i])
    return acc


TH = T // 2
CH0 = TH * 128


def _body(logits_hbm, legal_hbm, out_hbm, idx_v, fidx_v, vals_v, h0_v, h1_v,
          gsem, ssem0, ssem1, isem):
    wid = lax.axis_index("s") * NUM_CORES + lax.axis_index("c")
    row0 = wid * RPW

    idx_cp = pltpu.make_async_copy(legal_hbm.at[wid], idx_v, isem)
    idx_cp.start()

    idx_cp.wait()

    for k in range(KCH):
        base = (row0 + k // NCH) * A
        for i in range(CHUNK // LANES):
            sl = idx_v[k, pl.ds(i * LANES, LANES)]
            fidx_v[k, pl.ds(i * LANES, LANES)] = sl + base

    gcps = []
    for k in range(KCH):
        cp = pltpu.make_async_copy(logits_hbm.at[fidx_v.at[k]], vals_v.at[k], gsem)
        cp.start()
        gcps.append(cp)

    zvec = jnp.zeros((LANES,), jnp.float32)

    def _zero_half(h_v):
        def _zero_step(j, carry):
            for p in range(17):
                t = j * 17 + p
                for u in range(128 // LANES):
                    h_v[t, pl.ds(u * LANES, LANES)] = zvec
            return carry

        lax.fori_loop(0, TH // 17, _zero_step, 0)

    _zero_half(h0_v)

    for cp in gcps:
        cp.wait()

    for r in range(RPW):
        ks = range(r * NCH, (r + 1) * NCH)
        m = None
        for k in ks:
            for i in range(CHUNK // LANES):
                sl = vals_v[k, pl.ds(i * LANES, LANES)]
                m = sl if m is None else jnp.maximum(m, sl)
        mx = _red_scalar(m, jnp.maximum)
        s = jnp.zeros((LANES,), jnp.float32)
        for k in ks:
            for i in range(CHUNK // LANES):
                e = jnp.exp(vals_v[k, pl.ds(i * LANES, LANES)] - mx)
                vals_v[k, pl.ds(i * LANES, LANES)] = e
                s = s + e
        tot = _red_scalar(s, jnp.add)
        for k in ks:
            for i in range(CHUNK // LANES):
                vals_v[k, pl.ds(i * LANES, LANES)] = (
                    vals_v[k, pl.ds(i * LANES, LANES)] / tot)

    def _scatter_row(r, h_v, lo, restore):
        off = lo >> 7
        for k in range(r * NCH, (r + 1) * NCH):
            for i in range(CHUNK // LANES):
                ci = idx_v[k, pl.ds(i * LANES, LANES)]
                v = zvec if restore else vals_v[k, pl.ds(i * LANES, LANES)]
                msk = (ci >= lo) & (ci < lo + CH0)
                th = jnp.minimum(jnp.maximum((ci >> 7) - off, 0), TH - 1)
                plsc.store_scatter(h_v, [th, ci & 127], v, mask=msk)

    g0, rr0 = row0 >> 3, row0 & 7
    g1, rr1 = (row0 + 1) >> 3, (row0 + 1) & 7

    _scatter_row(0, h0_v, 0, False)
    s0a = pltpu.make_async_copy(h0_v, out_hbm.at[g0, pl.ds(0, TH), rr0, :],
                                ssem0)
    s0a.start()
    _zero_half(h1_v)
    _scatter_row(0, h1_v, CH0, False)
    s0b = pltpu.make_async_copy(h1_v, out_hbm.at[g0, pl.ds(TH, TH), rr0, :],
                                ssem1)
    s0b.start()
    s0a.wait()
    _scatter_row(0, h0_v, 0, True)
    _scatter_row(1, h0_v, 0, False)
    s1a = pltpu.make_async_copy(h0_v, out_hbm.at[g1, pl.ds(0, TH), rr1, :],
                                ssem0)
    s1a.start()
    s0b.wait()
    _scatter_row(0, h1_v, CH0, True)
    _scatter_row(1, h1_v, CH0, False)
    s1b = pltpu.make_async_copy(h1_v, out_hbm.at[g1, pl.ds(TH, TH), rr1, :],
                                ssem1)
    s1b.start()
    s1a.wait()
    s1b.wait()


def kernel(logits, legal_actions):
    mesh = plsc.VectorSubcoreMesh(core_axis_name="c", subcore_axis_name="s")
    run = pl.kernel(
        _body,
        mesh=mesh,
        compiler_params=pltpu.CompilerParams(needs_layout_passes=False),
        out_type=jax.ShapeDtypeStruct((G, T, 8, 128), jnp.float32),
        scratch_types=[
            pltpu.VMEM((KCH, CHUNK), jnp.int32),
            pltpu.VMEM((KCH, CHUNK), jnp.int32),
            pltpu.VMEM((KCH, CHUNK), jnp.float32),
            pltpu.VMEM((TH, 128), jnp.float32),
            pltpu.VMEM((TH, 128), jnp.float32),
            pltpu.SemaphoreType.DMA,
            pltpu.SemaphoreType.DMA,
            pltpu.SemaphoreType.DMA,
            pltpu.SemaphoreType.DMA,
        ],
    )
    out4 = run(logits.reshape(B * A), legal_actions.reshape(NW, KCH, CHUNK))
    out = out4.transpose(0, 2, 1, 3).reshape(B, AP)[:, :A]
    return out


# speedup vs baseline: 1.0091x; 1.0091x over previous
import jax
import jax.numpy as jnp
from jax import lax
from jax.experimental import pallas as pl
from jax.experimental.pallas import tpu as pltpu
from jax.experimental.pallas import tpu_sc as plsc

B = 64
A = 100000
L = 512
LANES = 16
NUM_CORES = 2
NUM_SUBCORES = 16
NW = NUM_CORES * NUM_SUBCORES
RPW = B // NW
CHUNK = 128
NCH = L // CHUNK
KCH = RPW * NCH
T = (A + 127) // 128
AP = T * 128
G = B // 8


def _red_scalar(vec, op):
    acc = vec[0]
    for i in range(1, LANES):
        acc = op(acc, vec[i])
    return acc


def _body(logits_hbm, legal_hbm, out_hbm, idx_v, fidx_v, vals_v, row_v,
          gsem, ssem, isem):
    wid = lax.axis_index("s") * NUM_CORES + lax.axis_index("c")
    row0 = wid * RPW

    idx_cp = pltpu.make_async_copy(legal_hbm.at[wid], idx_v, isem)
    idx_cp.start()

    idx_cp.wait()

    for k in range(KCH):
        base = (row0 + k // NCH) * A
        for i in range(CHUNK // LANES):
            sl = idx_v[k, pl.ds(i * LANES, LANES)]
            fidx_v[k, pl.ds(i * LANES, LANES)] = sl + base

    gcps = []
    for k in range(KCH):
        cp = pltpu.make_async_copy(logits_hbm.at[fidx_v.at[k]], vals_v.at[k], gsem)
        cp.start()
        gcps.append(cp)

    zvec = jnp.zeros((LANES,), jnp.float32)

    def _zero_step(j, carry):
        for p in range(2):
            t = j * 2 + p
            for u in range(128 // LANES):
                row_v[t, pl.ds(u * LANES, LANES)] = zvec
        return carry

    lax.fori_loop(0, T // 2, _zero_step, 0)

    for cp in gcps:
        cp.wait()

    for r in range(RPW):
        ks = range(r * NCH, (r + 1) * NCH)
        m = None
        for k in ks:
            for i in range(CHUNK // LANES):
                sl = vals_v[k, pl.ds(i * LANES, LANES)]
                m = sl if m is None else jnp.maximum(m, sl)
        mx = _red_scalar(m, jnp.maximum)
        s = jnp.zeros((LANES,), jnp.float32)
        for k in ks:
            for i in range(CHUNK // LANES):
                e = jnp.exp(vals_v[k, pl.ds(i * LANES, LANES)] - mx)
                vals_v[k, pl.ds(i * LANES, LANES)] = e
                s = s + e
        tot = _red_scalar(s, jnp.add)
        for k in ks:
            for i in range(CHUNK // LANES):
                vals_v[k, pl.ds(i * LANES, LANES)] = (
                    vals_v[k, pl.ds(i * LANES, LANES)] / tot)

    for r in range(RPW):
        row = row0 + r
        g = row >> 3
        rr = row & 7
        ks = range(r * NCH, (r + 1) * NCH)
        for k in ks:
            for i in range(CHUNK // LANES):
                ci = idx_v[k, pl.ds(i * LANES, LANES)]
                plsc.store_scatter(row_v, [ci >> 7, ci & 127],
                                   vals_v[k, pl.ds(i * LANES, LANES)])
        cp = pltpu.make_async_copy(row_v, out_hbm.at[g, :, rr, :], ssem)
        cp.start()
        cp.wait()
        if r + 1 < RPW:
            for k in ks:
                for i in range(CHUNK // LANES):
                    ci = idx_v[k, pl.ds(i * LANES, LANES)]
                    plsc.store_scatter(row_v, [ci >> 7, ci & 127], zvec)


def kernel(logits, legal_actions):
    mesh = plsc.VectorSubcoreMesh(core_axis_name="c", subcore_axis_name="s")
    run = pl.kernel(
        _body,
        mesh=mesh,
        compiler_params=pltpu.CompilerParams(needs_layout_passes=False),
        out_type=jax.ShapeDtypeStruct((G, T, 8, 128), jnp.float32),
        scratch_types=[
            pltpu.VMEM((KCH, CHUNK), jnp.int32),
            pltpu.VMEM((KCH, CHUNK), jnp.int32),
            pltpu.VMEM((KCH, CHUNK), jnp.float32),
            pltpu.VMEM((T, 128), jnp.float32),
            pltpu.SemaphoreType.DMA,
            pltpu.SemaphoreType.DMA,
            pltpu.SemaphoreType.DMA,
        ],
    )
    out4 = run(logits.reshape(B * A), legal_actions.reshape(NW, KCH, CHUNK))
    out = out4.transpose(0, 2, 1, 3).reshape(B, AP)[:, :A]
    return out
